# baseline (speedup 1.0000x reference)
import jax
import jax.numpy as jnp
from jax import lax
from jax.experimental import pallas as pl
from jax.experimental.pallas import tpu as pltpu

_N = 1024
_D = 512
_BLK = 256
_BIG = 2**30


def _body(x_all_ref, t_col_ref, t_row_ref, sq_col_ref, sq_row_ref,
          pair1_ref, pair2_ref, oh_ref):
    i = pl.program_id(0)
    h = pl.program_id(1)
    x_all = x_all_ref[...]
    x_blk = x_all_ref[pl.ds(i * _BLK, _BLK), :]
    pair1_ref[...] = x_blk

    @pl.when(h == 0)
    def _():
        t_col = t_col_ref[pl.ds(i * _BLK, _BLK), :]
        sq_col = sq_col_ref[pl.ds(i * _BLK, _BLK), :]
        sq_row = sq_row_ref[...]
        dot = lax.dot_general(x_blk, x_all, (((1,), (1,)), ((), ())),
                              preferred_element_type=jnp.float32)
        d2 = (sq_col + sq_row) - 2.0 * dot
        dist = jnp.sqrt(jnp.clip(d2, 1e-12, None))
        mask = t_col == t_row_ref[...]
        ids = lax.broadcasted_iota(jnp.int32, (_BLK, _N), 1)
        pos_d = jnp.where(mask, dist, -jnp.inf)
        pmax = jnp.max(pos_d, axis=1, keepdims=True)
        pos_idx = jnp.min(jnp.where(pos_d == pmax, ids, _BIG), axis=1,
                          keepdims=True)
        neg_d = jnp.where(mask, jnp.inf, dist)
        nmin = jnp.min(neg_d, axis=1, keepdims=True)
        neg_idx = jnp.min(jnp.where(neg_d == nmin, ids, _BIG), axis=1,
                          keepdims=True)
        onehot_p = (ids == pos_idx).astype(jnp.float32)
        oh_ref[...] = (ids == neg_idx).astype(jnp.float32)
        pair2_ref[...] = lax.dot_general(
            onehot_p, x_all, (((1,), (0,)), ((), ())),
            preferred_element_type=jnp.float32)

    @pl.when(h == 1)
    def _():
        pair2_ref[...] = lax.dot_general(
            oh_ref[...], x_all, (((1,), (0,)), ((), ())),
            preferred_element_type=jnp.float32)


def _fused(x, t_col, t_row, sq_col, sq_row, interpret=False):
    nblk = _N // _BLK
    return pl.pallas_call(
        _body,
        grid=(nblk, 2),
        in_specs=[
            pl.BlockSpec((_N, _D), lambda i, h: (0, 0)),
            pl.BlockSpec((_N, 1), lambda i, h: (0, 0)),
            pl.BlockSpec((1, _N), lambda i, h: (0, 0)),
            pl.BlockSpec((_N, 1), lambda i, h: (0, 0)),
            pl.BlockSpec((1, _N), lambda i, h: (0, 0)),
        ],
        out_specs=[
            pl.BlockSpec((_BLK, _D), lambda i, h: (h * nblk + i, 0)),
            pl.BlockSpec((_BLK, _D), lambda i, h: (h * nblk + i, 0)),
        ],
        out_shape=[
            jax.ShapeDtypeStruct((2 * _N, _D), jnp.float32),
            jax.ShapeDtypeStruct((2 * _N, _D), jnp.float32),
        ],
        scratch_shapes=[pltpu.VMEM((_BLK, _N), jnp.float32)],
        interpret=interpret,
    )(x, t_col, t_row, sq_col, sq_row)


def kernel(inputs, targets):
    t_col = targets.reshape(_N, 1)
    t_row = targets.reshape(1, _N)
    sq_col = jnp.sum(inputs * inputs, axis=1, keepdims=True)
    sq_row = sq_col.reshape(1, _N)
    pair1, pair2 = _fused(inputs, t_col, t_row, sq_col, sq_row)
    y = jnp.concatenate([jnp.ones((_N,), jnp.float32),
                         jnp.zeros((_N,), jnp.float32)])
    return (pair1, pair2, y)


# speedup vs baseline: 1.1668x; 1.1668x over previous
import jax
import jax.numpy as jnp
from jax import lax
from jax.experimental import pallas as pl
from jax.experimental.pallas import tpu as pltpu

_N = 1024
_D = 512
_BLK = 256
_BIG = 2**30


def _body(x_all_ref, t_col_ref, t_row_ref, sq_col_ref, sq_row_ref,
          pair1_ref, pair2_ref):
    i = pl.program_id(0)
    x_all = x_all_ref[...]
    x_blk = x_all_ref[pl.ds(i * _BLK, _BLK), :]
    t_col = t_col_ref[pl.ds(i * _BLK, _BLK), :]
    sq_col = sq_col_ref[pl.ds(i * _BLK, _BLK), :]
    sq_row = sq_row_ref[...]
    t_row = t_row_ref[...]

    dot = lax.dot_general(x_blk, x_all, (((1,), (1,)), ((), ())),
                          preferred_element_type=jnp.float32)
    d2 = (sq_col + sq_row) - 2.0 * dot
    dist = jnp.sqrt(jnp.clip(d2, 1e-12, None))
    mask = t_col == t_row
    ids = lax.broadcasted_iota(jnp.int32, (_BLK, _N), 1)
    pos_d = jnp.where(mask, dist, -jnp.inf)
    pmax = jnp.max(pos_d, axis=1, keepdims=True)
    pos_idx = jnp.min(jnp.where(pos_d == pmax, ids, _BIG), axis=1,
                      keepdims=True)
    neg_d = jnp.where(mask, jnp.inf, dist)
    nmin = jnp.min(neg_d, axis=1, keepdims=True)
    neg_idx = jnp.min(jnp.where(neg_d == nmin, ids, _BIG), axis=1,
                      keepdims=True)
    onehot_p = (ids == pos_idx).astype(jnp.float32)
    onehot_n = (ids == neg_idx).astype(jnp.float32)
    pair2_ref[0] = lax.dot_general(onehot_p, x_all, (((1,), (0,)), ((), ())),
                                   preferred_element_type=jnp.float32)
    pair2_ref[1] = lax.dot_general(onehot_n, x_all, (((1,), (0,)), ((), ())),
                                   preferred_element_type=jnp.float32)
    pair1_ref[0] = x_blk
    pair1_ref[1] = x_blk


def _fused(x, t_col, t_row, sq_col, sq_row, interpret=False):
    return pl.pallas_call(
        _body,
        grid=(_N // _BLK,),
        in_specs=[
            pl.BlockSpec((_N, _D), lambda i: (0, 0)),
            pl.BlockSpec((_N, 1), lambda i: (0, 0)),
            pl.BlockSpec((1, _N), lambda i: (0, 0)),
            pl.BlockSpec((_N, 1), lambda i: (0, 0)),
            pl.BlockSpec((1, _N), lambda i: (0, 0)),
        ],
        out_specs=[
            pl.BlockSpec((2, _BLK, _D), lambda i: (0, i, 0)),
            pl.BlockSpec((2, _BLK, _D), lambda i: (0, i, 0)),
        ],
        out_shape=[
            jax.ShapeDtypeStruct((2, _N, _D), jnp.float32),
            jax.ShapeDtypeStruct((2, _N, _D), jnp.float32),
        ],
        interpret=interpret,
    )(x, t_col, t_row, sq_col, sq_row)


def kernel(inputs, targets):
    t_col = targets.reshape(_N, 1)
    t_row = targets.reshape(1, _N)
    sq_col = jnp.sum(inputs * inputs, axis=1, keepdims=True)
    sq_row = sq_col.reshape(1, _N)
    pair1, pair2 = _fused(inputs, t_col, t_row, sq_col, sq_row)
    y = jnp.concatenate([jnp.ones((_N,), jnp.float32),
                         jnp.zeros((_N,), jnp.float32)])
    return (pair1.reshape(2 * _N, _D), pair2.reshape(2 * _N, _D), y)


# speedup vs baseline: 1.2281x; 1.0526x over previous
import jax
import jax.numpy as jnp
from jax import lax
from jax.experimental import pallas as pl
from jax.experimental.pallas import tpu as pltpu

_N = 1024
_D = 512
_BLK = 512
_BIG = 2**30


def _body(x_all_ref, t_col_ref, t_row_ref, sq_col_ref, sq_row_ref,
          pair1_ref, pair2_ref):
    i = pl.program_id(0)
    x_all = x_all_ref[...]
    x_blk = x_all_ref[pl.ds(i * _BLK, _BLK), :]
    t_col = t_col_ref[pl.ds(i * _BLK, _BLK), :]
    sq_col = sq_col_ref[pl.ds(i * _BLK, _BLK), :]
    sq_row = sq_row_ref[...]
    t_row = t_row_ref[...]

    dot = lax.dot_general(x_blk, x_all, (((1,), (1,)), ((), ())),
                          preferred_element_type=jnp.float32)
    d2 = (sq_col + sq_row) - 2.0 * dot
    dist = jnp.sqrt(jnp.clip(d2, 1e-12, None))
    mask = t_col == t_row
    ids = lax.broadcasted_iota(jnp.int32, (_BLK, _N), 1)
    pos_d = jnp.where(mask, dist, -jnp.inf)
    pmax = jnp.max(pos_d, axis=1, keepdims=True)
    pos_idx = jnp.min(jnp.where(pos_d == pmax, ids, _BIG), axis=1,
                      keepdims=True)
    neg_d = jnp.where(mask, jnp.inf, dist)
    nmin = jnp.min(neg_d, axis=1, keepdims=True)
    neg_idx = jnp.min(jnp.where(neg_d == nmin, ids, _BIG), axis=1,
                      keepdims=True)
    onehot_p = (ids == pos_idx).astype(jnp.float32)
    onehot_n = (ids == neg_idx).astype(jnp.float32)
    pair2_ref[0] = lax.dot_general(onehot_p, x_all, (((1,), (0,)), ((), ())),
                                   preferred_element_type=jnp.float32)
    pair2_ref[1] = lax.dot_general(onehot_n, x_all, (((1,), (0,)), ((), ())),
                                   preferred_element_type=jnp.float32)
    pair1_ref[0] = x_blk
    pair1_ref[1] = x_blk


def _fused(x, t_col, t_row, sq_col, sq_row, interpret=False):
    return pl.pallas_call(
        _body,
        grid=(_N // _BLK,),
        in_specs=[
            pl.BlockSpec((_N, _D), lambda i: (0, 0)),
            pl.BlockSpec((_N, 1), lambda i: (0, 0)),
            pl.BlockSpec((1, _N), lambda i: (0, 0)),
            pl.BlockSpec((_N, 1), lambda i: (0, 0)),
            pl.BlockSpec((1, _N), lambda i: (0, 0)),
        ],
        out_specs=[
            pl.BlockSpec((2, _BLK, _D), lambda i: (0, i, 0)),
            pl.BlockSpec((2, _BLK, _D), lambda i: (0, i, 0)),
        ],
        out_shape=[
            jax.ShapeDtypeStruct((2, _N, _D), jnp.float32),
            jax.ShapeDtypeStruct((2, _N, _D), jnp.float32),
        ],
        interpret=interpret,
    )(x, t_col, t_row, sq_col, sq_row)


def kernel(inputs, targets):
    t_col = targets.reshape(_N, 1)
    t_row = targets.reshape(1, _N)
    sq_col = jnp.sum(inputs * inputs, axis=1, keepdims=True)
    sq_row = sq_col.reshape(1, _N)
    pair1, pair2 = _fused(inputs, t_col, t_row, sq_col, sq_row)
    y = jnp.concatenate([jnp.ones((_N,), jnp.float32),
                         jnp.zeros((_N,), jnp.float32)])
    return (pair1.reshape(2 * _N, _D), pair2.reshape(2 * _N, _D), y)


# speedup vs baseline: 1.2583x; 1.0246x over previous
import jax
import jax.numpy as jnp
from jax import lax
from jax.experimental import pallas as pl
from jax.experimental.pallas import tpu as pltpu

_N = 1024
_D = 512
_BLK = 1024
_BIG = 2**30


def _body(x_all_ref, t_col_ref, t_row_ref, sq_col_ref, sq_row_ref,
          pair1_ref, pair2_ref):
    i = pl.program_id(0)
    x_all = x_all_ref[...]
    x_blk = x_all_ref[pl.ds(i * _BLK, _BLK), :]
    t_col = t_col_ref[pl.ds(i * _BLK, _BLK), :]
    sq_col = sq_col_ref[pl.ds(i * _BLK, _BLK), :]
    sq_row = sq_row_ref[...]
    t_row = t_row_ref[...]

    dot = lax.dot_general(x_blk, x_all, (((1,), (1,)), ((), ())),
                          preferred_element_type=jnp.float32)
    d2 = (sq_col + sq_row) - 2.0 * dot
    dist = jnp.sqrt(jnp.clip(d2, 1e-12, None))
    mask = t_col == t_row
    ids = lax.broadcasted_iota(jnp.int32, (_BLK, _N), 1)
    pos_d = jnp.where(mask, dist, -jnp.inf)
    pmax = jnp.max(pos_d, axis=1, keepdims=True)
    pos_idx = jnp.min(jnp.where(pos_d == pmax, ids, _BIG), axis=1,
                      keepdims=True)
    neg_d = jnp.where(mask, jnp.inf, dist)
    nmin = jnp.min(neg_d, axis=1, keepdims=True)
    neg_idx = jnp.min(jnp.where(neg_d == nmin, ids, _BIG), axis=1,
                      keepdims=True)
    onehot_p = (ids == pos_idx).astype(jnp.float32)
    onehot_n = (ids == neg_idx).astype(jnp.float32)
    pair2_ref[0] = lax.dot_general(onehot_p, x_all, (((1,), (0,)), ((), ())),
                                   preferred_element_type=jnp.float32)
    pair2_ref[1] = lax.dot_general(onehot_n, x_all, (((1,), (0,)), ((), ())),
                                   preferred_element_type=jnp.float32)
    pair1_ref[0] = x_blk
    pair1_ref[1] = x_blk


def _fused(x, t_col, t_row, sq_col, sq_row, interpret=False):
    return pl.pallas_call(
        _body,
        grid=(_N // _BLK,),
        in_specs=[
            pl.BlockSpec((_N, _D), lambda i: (0, 0)),
            pl.BlockSpec((_N, 1), lambda i: (0, 0)),
            pl.BlockSpec((1, _N), lambda i: (0, 0)),
            pl.BlockSpec((_N, 1), lambda i: (0, 0)),
            pl.BlockSpec((1, _N), lambda i: (0, 0)),
        ],
        out_specs=[
            pl.BlockSpec((2, _BLK, _D), lambda i: (0, i, 0)),
            pl.BlockSpec((2, _BLK, _D), lambda i: (0, i, 0)),
        ],
        out_shape=[
            jax.ShapeDtypeStruct((2, _N, _D), jnp.float32),
            jax.ShapeDtypeStruct((2, _N, _D), jnp.float32),
        ],
        interpret=interpret,
    )(x, t_col, t_row, sq_col, sq_row)


def kernel(inputs, targets):
    t_col = targets.reshape(_N, 1)
    t_row = targets.reshape(1, _N)
    sq_col = jnp.sum(inputs * inputs, axis=1, keepdims=True)
    sq_row = sq_col.reshape(1, _N)
    pair1, pair2 = _fused(inputs, t_col, t_row, sq_col, sq_row)
    y = jnp.concatenate([jnp.ones((_N,), jnp.float32),
                         jnp.zeros((_N,), jnp.float32)])
    return (pair1.reshape(2 * _N, _D), pair2.reshape(2 * _N, _D), y)
